# Initial kernel scaffold; baseline (speedup 1.0000x reference)
import jax
import jax.numpy as jnp
from jax.experimental import pallas as pl


def kernel(user_feat, item_feat, u2i_edge_index, i2u_edge_index):
    raise NotImplementedError("write your pallas kernel here")


# speedup vs baseline: 2.3500x; 2.3500x over previous
import functools

import jax
import jax.numpy as jnp
from jax import lax
from jax.experimental import pallas as pl
from jax.experimental.pallas import tpu as pltpu
from jax.experimental.pallas import tpu_sc as plsc

D = 128
D2 = 64
CH = 128
NS = 16
LANES = 16


def _norm_tc_body(uf, itf, un_lo, un_hi, in_lo, in_hi, us, isc):
    x = uf[...]
    s2 = jnp.sum(x * x, axis=1)
    us[...] = jnp.sqrt(s2)
    xn = x * lax.rsqrt(s2)[:, None]
    un_lo[...] = xn[:, :D2]
    un_hi[...] = xn[:, D2:]
    y = itf[...]
    t2 = jnp.sum(y * y, axis=1)
    isc[...] = jnp.sqrt(t2)
    yn = y * lax.rsqrt(t2)[:, None]
    in_lo[...] = yn[:, :D2]
    in_hi[...] = yn[:, D2:]


def _epi_tc_body(n_u, n_i, ul, uh, ud, il, ih, idg, out_u, out_i):
    ru = lax.rsqrt(jnp.maximum(ud[...][:n_u], 1.0))[:, None]
    out_u[:, :D2] = ul[...][:n_u] * ru
    out_u[:, D2:] = uh[...][:n_u] * ru
    ri = lax.rsqrt(jnp.maximum(idg[...][:n_i], 1.0))[:, None]
    out_i[:, :D2] = il[...][:n_i] * ri
    out_i[:, D2:] = ih[...][:n_i] * ri


def _hsum(v):
    idx = lax.iota(jnp.int32, LANES)
    for sh in (8, 4, 2, 1):
        rot = v.at[(idx + sh) & (LANES - 1)].get(mode="promise_in_bounds")
        v = v + rot
    return v[0]


def _zero2d(ref, nrows, ncols):
    def row(i, _):
        for k in range(ncols // LANES):
            ref[i, pl.ds(k * LANES, LANES)] = jnp.zeros((LANES,), jnp.float32)
        return 0

    lax.fori_loop(0, nrows, row, 0)


def _zero_acc_stripe(zsrc, sp_acc, srow, stripe):
    nfull, rem = stripe // CH, stripe % CH
    for j in range(nfull):
        pltpu.sync_copy(zsrc, sp_acc.at[pl.ds(srow + j * CH, CH)])
    if rem:
        pltpu.sync_copy(zsrc.at[pl.ds(0, rem)],
                        sp_acc.at[pl.ds(srow + nfull * CH, rem)])


def _conv_sc(sid, nchunk, stripe, norm_lo, norm_hi, scale_hbm, src_t, dst_t,
             mean_lo, mean_hi, deg_hbm, out_lo, out_hi,
             idx_s, idx_d, a_v, b_v, c_v, dot_v, w_all, m8_v, d8_v,
             ones_v, sp_acc, sp_deg):
    srow = sid * stripe
    f32 = jnp.float32

    pltpu.sync_copy(src_t.at[sid], idx_s)
    pltpu.sync_copy(dst_t.at[sid], idx_d)
    _zero2d(a_v, CH, D2)
    _zero_acc_stripe(a_v, sp_acc, srow, stripe)
    for k in range(CH // LANES):
        c_v[pl.ds(k * LANES, LANES)] = jnp.zeros((LANES,), f32)
    nfull, rem = stripe // CH, stripe % CH
    for j in range(nfull):
        pltpu.sync_copy(c_v, sp_deg.at[pl.ds(srow + j * CH, CH)])
    if rem:
        pltpu.sync_copy(c_v.at[pl.ds(0, rem)],
                        sp_deg.at[pl.ds(srow + nfull * CH, rem)])
    plsc.subcore_barrier()

    def seg_sum_half(norm_h, with_deg):
        def phase_b(j, _):
            pltpu.sync_copy(norm_h.at[idx_s.at[j]], a_v)
            pltpu.sync_copy(a_v, sp_acc.at[idx_d.at[j]], add=True)
            if with_deg:
                pltpu.sync_copy(ones_v, sp_deg.at[idx_d.at[j]], add=True)
            return 0

        lax.fori_loop(0, nchunk, phase_b, 0)
        plsc.subcore_barrier()

    def dump_mean_half(mean_h, with_deg):
        def mean_div(j, _):
            r0 = srow + j * 8
            pltpu.sync_copy(sp_acc.at[pl.ds(r0, 8)], m8_v)
            pltpu.sync_copy(sp_deg.at[pl.ds(r0, 8)], d8_v.at[pl.ds(0, 8)])
            if with_deg:
                pltpu.sync_copy(d8_v.at[pl.ds(0, 8)],
                                deg_hbm.at[pl.ds(r0, 8)])
            dv = 1.0 / jnp.maximum(d8_v[...], 1.0)
            for r in range(8):
                w = dv[r]
                for k in range(D2 // LANES):
                    sl = pl.ds(k * LANES, LANES)
                    m8_v[r, sl] = m8_v[r, sl] * w
            pltpu.sync_copy(m8_v, mean_h.at[pl.ds(r0, 8)])
            return 0

        lax.fori_loop(0, stripe // 8, mean_div, 0)
        _zero2d(a_v, CH, D2)
        _zero_acc_stripe(a_v, sp_acc, srow, stripe)
        plsc.subcore_barrier()

    def dump_out_half(out_h, rezero):
        def cp(r0, n, buf):
            pltpu.sync_copy(sp_acc.at[pl.ds(r0, n)], buf)
            pltpu.sync_copy(buf, out_h.at[pl.ds(r0, n)])

        for j in range(nfull):
            cp(srow + j * CH, CH, a_v)
        if rem:
            cp(srow + nfull * CH, rem, a_v.at[pl.ds(0, rem)])
        if rezero:
            _zero2d(a_v, CH, D2)
            _zero_acc_stripe(a_v, sp_acc, srow, stripe)
        plsc.subcore_barrier()

    seg_sum_half(norm_lo, True)
    dump_mean_half(mean_lo, True)
    seg_sum_half(norm_hi, False)
    dump_mean_half(mean_hi, False)

    lane = lax.iota(jnp.int32, LANES)

    def phase_c12(j, _):
        pltpu.sync_copy(mean_lo.at[idx_d.at[j]], a_v)
        pltpu.sync_copy(norm_lo.at[idx_s.at[j]], b_v)

        def group_lo(g, _):
            d16 = jnp.zeros((LANES,), jnp.float32)
            for t in range(LANES):
                i = g * LANES + t
                acc = a_v[i, pl.ds(0, LANES)] * b_v[i, pl.ds(0, LANES)]
                for k in range(1, D2 // LANES):
                    sl = pl.ds(k * LANES, LANES)
                    acc = acc + a_v[i, sl] * b_v[i, sl]
                d16 = jnp.where(lane == t, _hsum(acc), d16)
            dot_v[pl.ds(g * LANES, LANES)] = d16
            return 0

        lax.fori_loop(0, CH // LANES, group_lo, 0)

        pltpu.sync_copy(mean_hi.at[idx_d.at[j]], a_v)
        pltpu.sync_copy(norm_hi.at[idx_s.at[j]], b_v)
        pltpu.sync_copy(scale_hbm.at[idx_s.at[j]], c_v)

        def group_hi(g, _):
            cw = c_v[pl.ds(g * LANES, LANES)]
            d16 = dot_v[pl.ds(g * LANES, LANES)]
            w16 = jnp.zeros((LANES,), jnp.float32)
            for t in range(LANES):
                i = g * LANES + t
                acc = a_v[i, pl.ds(0, LANES)] * b_v[i, pl.ds(0, LANES)]
                for k in range(1, D2 // LANES):
                    sl = pl.ds(k * LANES, LANES)
                    acc = acc + a_v[i, sl] * b_v[i, sl]
                dot = _hsum(acc) + d16[t]
                w = jnp.maximum(dot, 0.0) * cw[t]
                w16 = jnp.where(lane == t, w, w16)
                for k in range(D2 // LANES):
                    sl = pl.ds(k * LANES, LANES)
                    b_v[i, sl] = b_v[i, sl] * w
            w_all[j, pl.ds(g * LANES, LANES)] = w16
            return 0

        lax.fori_loop(0, CH // LANES, group_hi, 0)
        pltpu.sync_copy(b_v, sp_acc.at[idx_d.at[j]], add=True)
        return 0

    lax.fori_loop(0, nchunk, phase_c12, 0)
    plsc.subcore_barrier()
    dump_out_half(out_hi, True)

    def phase_c3(j, _):
        pltpu.sync_copy(norm_lo.at[idx_s.at[j]], b_v)

        def group(g, _):
            w16 = w_all[j, pl.ds(g * LANES, LANES)]
            for t in range(LANES):
                i = g * LANES + t
                w = w16[t]
                for k in range(D2 // LANES):
                    sl = pl.ds(k * LANES, LANES)
                    b_v[i, sl] = b_v[i, sl] * w
            return 0

        lax.fori_loop(0, CH // LANES, group, 0)
        pltpu.sync_copy(b_v, sp_acc.at[idx_d.at[j]], add=True)
        return 0

    lax.fori_loop(0, nchunk, phase_c3, 0)
    plsc.subcore_barrier()
    dump_out_half(out_lo, False)


def _pad_edges(eidx, nchunk, pad_row):
    e = eidx.shape[1]
    ept = nchunk * CH
    pad = NS * ept - e
    src = jnp.concatenate(
        [eidx[0].astype(jnp.int32), jnp.zeros((pad,), jnp.int32)])
    dst = jnp.concatenate(
        [eidx[1].astype(jnp.int32), jnp.full((pad,), pad_row, jnp.int32)])
    return src.reshape(NS, nchunk, CH), dst.reshape(NS, nchunk, CH)


def kernel(user_feat, item_feat, u2i_edge_index, i2u_edge_index):
    n_user, d = user_feat.shape
    n_item = item_feat.shape[0]
    assert d == D
    e = u2i_edge_index.shape[1]
    nchunk = -(-e // (NS * CH))
    n_max = max(n_user, n_item)
    npad = -(-(n_max + 1) // (NS * 8)) * (NS * 8)
    stripe = npad // NS

    f32 = jnp.float32
    (user_lo, user_hi, item_lo, item_hi, user_scale, item_scale) = (
        pl.pallas_call(
            _norm_tc_body,
            out_shape=[
                jax.ShapeDtypeStruct((n_user, D2), f32),
                jax.ShapeDtypeStruct((n_user, D2), f32),
                jax.ShapeDtypeStruct((n_item, D2), f32),
                jax.ShapeDtypeStruct((n_item, D2), f32),
                jax.ShapeDtypeStruct((n_user,), f32),
                jax.ShapeDtypeStruct((n_item,), f32),
            ],
        )(user_feat, item_feat))

    u2i_src, u2i_dst = _pad_edges(u2i_edge_index, nchunk, n_item)
    i2u_src, i2u_dst = _pad_edges(i2u_edge_index, nchunk, n_user)
    ones = jnp.ones((CH,), f32)

    mesh = plsc.VectorSubcoreMesh(core_axis_name="c", subcore_axis_name="s")

    sds = jax.ShapeDtypeStruct

    @functools.partial(
        pl.kernel,
        mesh=mesh,
        compiler_params=pltpu.CompilerParams(use_tc_tiling_on_sc=False),
        out_type=[
            sds((npad, D2), f32),
            sds((npad, D2), f32),
            sds((npad, D2), f32),
            sds((npad, D2), f32),
            sds((npad,), f32),
            sds((npad,), f32),
            sds((npad, D2), f32),
            sds((npad, D2), f32),
            sds((npad, D2), f32),
            sds((npad, D2), f32),
        ],
        scratch_types=[
            pltpu.VMEM((nchunk, CH), jnp.int32),
            pltpu.VMEM((nchunk, CH), jnp.int32),
            pltpu.VMEM((CH, D2), f32),
            pltpu.VMEM((CH, D2), f32),
            pltpu.VMEM((CH,), f32),
            pltpu.VMEM((CH,), f32),
            pltpu.VMEM((nchunk, CH), f32),
            pltpu.VMEM((8, D2), f32),
            pltpu.VMEM((LANES,), f32),
            pltpu.VMEM((CH,), f32),
            pltpu.VMEM_SHARED((npad, D2), f32),
            pltpu.VMEM_SHARED((npad,), f32),
        ],
    )
    def sc_main(user_lo_h, user_hi_h, item_lo_h, item_hi_h,
                user_scale_h, item_scale_h,
                u2i_src_h, u2i_dst_h, i2u_src_h, i2u_dst_h, ones_h,
                item_out_lo, item_out_hi, user_out_lo, user_out_hi,
                item_deg_h, user_deg_h,
                item_mean_lo, item_mean_hi, user_mean_lo, user_mean_hi,
                idx_s, idx_d, a_v, b_v, c_v, dot_v, w_all, m8_v, d8_v,
                ones_v, sp_acc, sp_deg):
        cid = lax.axis_index("c")
        sid = lax.axis_index("s")
        pltpu.sync_copy(ones_h, ones_v)

        @pl.when(cid == 0)
        def _():
            _conv_sc(sid, nchunk, stripe, user_lo_h, user_hi_h,
                     user_scale_h, u2i_src_h, u2i_dst_h,
                     item_mean_lo, item_mean_hi, item_deg_h,
                     item_out_lo, item_out_hi,
                     idx_s, idx_d, a_v, b_v, c_v, dot_v, w_all,
                     m8_v, d8_v, ones_v, sp_acc, sp_deg)

        @pl.when(cid == 1)
        def _():
            _conv_sc(sid, nchunk, stripe, item_lo_h, item_hi_h,
                     item_scale_h, i2u_src_h, i2u_dst_h,
                     user_mean_lo, user_mean_hi, user_deg_h,
                     user_out_lo, user_out_hi,
                     idx_s, idx_d, a_v, b_v, c_v, dot_v, w_all,
                     m8_v, d8_v, ones_v, sp_acc, sp_deg)

    (item_out_lo, item_out_hi, user_out_lo, user_out_hi,
     item_deg, user_deg, _, _, _, _) = sc_main(
        user_lo, user_hi, item_lo, item_hi, user_scale, item_scale,
        u2i_src, u2i_dst, i2u_src, i2u_dst, ones)

    user_out, item_out = pl.pallas_call(
        functools.partial(_epi_tc_body, n_user, n_item),
        out_shape=[
            jax.ShapeDtypeStruct((n_user, D), f32),
            jax.ShapeDtypeStruct((n_item, D), f32),
        ],
    )(user_out_lo, user_out_hi, user_deg,
      item_out_lo, item_out_hi, item_deg)
    return (user_out, item_out)
